# baseline (speedup 1.0000x reference)
import functools

import jax
import jax.numpy as jnp
from jax import lax
from jax.experimental import pallas as pl
from jax.experimental.pallas import tpu as pltpu
from jax.experimental.pallas import tpu_sc as plsc

_VOCAB = 1000
_EMB = 32
_T = 8
_B = 4096
_NROWS = _B * _T
_VPAD = 1024

_NW = 32
_ROWS_PER_W = _NROWS // _NW
_CHUNK = 16
_NCHUNK = _ROWS_PER_W // _CHUNK
_BPC = _CHUNK // _T
_NG = 4
_NO = 2

_NVREG = _VOCAB // 16
_TAILO = _NVREG * 16
_RU = 4


def _table_kernel(emb_ref, posw_ref, w_ref, out_ref):
    t = pl.program_id(0)
    ew = jnp.dot(emb_ref[:], w_ref[:], preferred_element_type=jnp.float32)
    out_ref[:] = ew + posw_ref[t, :][None, :]


def _build_table8(embedding, positional_embedding, lm_head_w, lm_head_b):
    posw = positional_embedding @ lm_head_w + lm_head_b[None, :]
    posw = jnp.pad(posw, ((0, 0), (0, _VPAD - _VOCAB)))
    w_pad = jnp.pad(lm_head_w, ((0, 0), (0, _VPAD - _VOCAB)))
    return pl.pallas_call(
        _table_kernel,
        grid=(_T,),
        in_specs=[
            pl.BlockSpec((_VOCAB, _EMB), lambda t: (0, 0)),
            pl.BlockSpec((_T, _VPAD), lambda t: (0, 0)),
            pl.BlockSpec((_EMB, _VPAD), lambda t: (0, 0)),
        ],
        out_specs=pl.BlockSpec((_VOCAB, _VPAD), lambda t: (t, 0)),
        out_shape=jax.ShapeDtypeStruct((_T * _VOCAB, _VPAD), jnp.float32),
    )(embedding, posw, w_pad)


def _repack_chunk(bufg, bufo):
    lanes = lax.iota(jnp.int32, 16)
    tail_lane = _TAILO + lanes
    tail_mask = lanes < (_VOCAB - _TAILO)
    zeros = jnp.zeros((16,), jnp.int32)

    def quad(q, _):
        r0 = _RU * q
        for i in range(_RU):
            r = r0 + i
            b_loc = r // _T
            t = r % _T
            for j in range(_NVREG):
                o = 16 * j
                bufo[b_loc, t, pl.ds(o, 16)] = bufg[r, pl.ds(o, 16)]
            plsc.store_scatter(
                bufo, [zeros + b_loc, zeros + t, tail_lane],
                bufg[r, pl.ds(_TAILO, 16)], mask=tail_mask)
        return ()

    lax.fori_loop(0, _CHUNK // _RU, quad, ())


def _sc_gather_body(tab_ref, jidx_ref, out_ref, idx_v,
                    bufg0, bufg1, bufg2, bufg3, bufo0, bufo1,
                    gsem0, gsem1, gsem2, gsem3, osem0, osem1):
    wid = lax.axis_index("s") * 2 + lax.axis_index("c")
    base_b = wid * (_ROWS_PER_W // _T)

    pltpu.sync_copy(jidx_ref.at[wid], idx_v)

    bufgs = (bufg0, bufg1, bufg2, bufg3)
    bufos = (bufo0, bufo1)
    gsems = (gsem0, gsem1, gsem2, gsem3)
    osems = (osem0, osem1)

    def gather(c, g):
        pltpu.async_copy(tab_ref.at[idx_v.at[c]], bufgs[g], gsems[g])

    def wait_gather(g):
        pltpu.make_async_copy(tab_ref.at[pl.ds(0, _CHUNK)],
                              bufgs[g], gsems[g]).wait()

    def out_copy(c, p):
        pltpu.async_copy(
            bufos[p],
            out_ref.at[pl.ds(base_b + c * _BPC, _BPC)],
            osems[p],
        )

    def wait_out(p):
        pltpu.make_async_copy(out_ref.at[pl.ds(0, _BPC)],
                              bufos[p], osems[p]).wait()

    for g in range(_NG):
        gather(g, g)

    def body(k, _):
        c0 = _NG * k
        for g in range(_NG):
            c = c0 + g
            p = g & 1
            wait_gather(g)
            pl.when(c >= _NO)(lambda: wait_out(p))
            _repack_chunk(bufgs[g], bufos[p])
            pl.when(c + _NG < _NCHUNK)(lambda: gather(c + _NG, g))
            out_copy(c, p)
        return ()

    lax.fori_loop(0, _NCHUNK // _NG, body, ())
    wait_out(0)
    wait_out(1)


def _sc_gather(table8, jidx):
    mesh = plsc.VectorSubcoreMesh(core_axis_name="c", subcore_axis_name="s")
    fn = functools.partial(
        pl.kernel,
        out_type=jax.ShapeDtypeStruct((_B, _T, _VOCAB), jnp.float32),
        mesh=mesh,
        scratch_types=[
            pltpu.VMEM((_NCHUNK, _CHUNK), jnp.int32),
            pltpu.VMEM((_CHUNK, _VPAD), jnp.float32),
            pltpu.VMEM((_CHUNK, _VPAD), jnp.float32),
            pltpu.VMEM((_CHUNK, _VPAD), jnp.float32),
            pltpu.VMEM((_CHUNK, _VPAD), jnp.float32),
            pltpu.VMEM((_BPC, _T, _VOCAB), jnp.float32),
            pltpu.VMEM((_BPC, _T, _VOCAB), jnp.float32),
            pltpu.SemaphoreType.DMA,
            pltpu.SemaphoreType.DMA,
            pltpu.SemaphoreType.DMA,
            pltpu.SemaphoreType.DMA,
            pltpu.SemaphoreType.DMA,
            pltpu.SemaphoreType.DMA,
        ],
        compiler_params=pltpu.CompilerParams(needs_layout_passes=False),
    )(_sc_gather_body)
    return fn(table8, jidx)


@jax.jit
def kernel(idx, embedding, positional_embedding, lm_head_w, lm_head_b):
    table8 = _build_table8(embedding, positional_embedding, lm_head_w,
                           lm_head_b)
    flat = idx.reshape(_NROWS).astype(jnp.int32)
    j = flat + _VOCAB * (jax.lax.iota(jnp.int32, _NROWS) % _T)
    jidx = j.reshape(_NW, _NCHUNK, _CHUNK)
    return _sc_gather(table8, jidx)


# speedup vs baseline: 1.5384x; 1.5384x over previous
import functools

import jax
import jax.numpy as jnp
from jax import lax
from jax.experimental import pallas as pl
from jax.experimental.pallas import tpu as pltpu
from jax.experimental.pallas import tpu_sc as plsc

_VOCAB = 1000
_EMB = 32
_T = 8
_B = 4096
_NROWS = _B * _T
_VPAD = 1024

_NW = 32
_ROWS_PER_W = _NROWS // _NW
_CHUNK = 16
_NCHUNK = _ROWS_PER_W // _CHUNK
_BPC = _CHUNK // _T
_NG = 4
_NO = 2

_NVREG = _VOCAB // 16
_TAILO = _NVREG * 16
_RU = 8


def _table_kernel(emb_ref, posw_ref, w_ref, out_ref):
    t = pl.program_id(0)
    ew = jnp.dot(emb_ref[:], w_ref[:], preferred_element_type=jnp.float32)
    out_ref[:] = ew + posw_ref[t, :][None, :]


def _build_table8(embedding, positional_embedding, lm_head_w, lm_head_b):
    posw = positional_embedding @ lm_head_w + lm_head_b[None, :]
    posw = jnp.pad(posw, ((0, 0), (0, _VPAD - _VOCAB)))
    w_pad = jnp.pad(lm_head_w, ((0, 0), (0, _VPAD - _VOCAB)))
    return pl.pallas_call(
        _table_kernel,
        grid=(_T,),
        in_specs=[
            pl.BlockSpec((_VOCAB, _EMB), lambda t: (0, 0)),
            pl.BlockSpec((_T, _VPAD), lambda t: (0, 0)),
            pl.BlockSpec((_EMB, _VPAD), lambda t: (0, 0)),
        ],
        out_specs=pl.BlockSpec((_VOCAB, _VPAD), lambda t: (t, 0)),
        out_shape=jax.ShapeDtypeStruct((_T * _VOCAB, _VPAD), jnp.float32),
    )(embedding, posw, w_pad)


def _repack_chunk(bufg, bufo):
    lanes = lax.iota(jnp.int32, 16)
    tail_lane = _TAILO + lanes
    tail_mask = lanes < (_VOCAB - _TAILO)
    zeros = jnp.zeros((16,), jnp.int32)

    def quad(q, _):
        r0 = _RU * q
        for i in range(_RU):
            r = r0 + i
            b_loc = r // _T
            t = r % _T
            for j in range(_NVREG):
                o = 16 * j
                bufo[b_loc, t, pl.ds(o, 16)] = bufg[r, pl.ds(o, 16)]
            plsc.store_scatter(
                bufo, [zeros + b_loc, zeros + t, tail_lane],
                bufg[r, pl.ds(_TAILO, 16)], mask=tail_mask)
        return ()

    lax.fori_loop(0, _CHUNK // _RU, quad, ())


def _sc_gather_body(tab_ref, jidx_ref, out_ref, idx_v,
                    bufg0, bufg1, bufg2, bufg3, bufo0, bufo1,
                    gsem0, gsem1, gsem2, gsem3, osem0, osem1):
    wid = lax.axis_index("s") * 2 + lax.axis_index("c")
    base_b = wid * (_ROWS_PER_W // _T)

    pltpu.sync_copy(jidx_ref.at[wid], idx_v)

    bufgs = (bufg0, bufg1, bufg2, bufg3)
    bufos = (bufo0, bufo1)
    gsems = (gsem0, gsem1, gsem2, gsem3)
    osems = (osem0, osem1)

    def gather(c, g):
        pltpu.async_copy(tab_ref.at[idx_v.at[c]], bufgs[g], gsems[g])

    def wait_gather(g):
        pltpu.make_async_copy(tab_ref.at[pl.ds(0, _CHUNK)],
                              bufgs[g], gsems[g]).wait()

    def out_copy(c, p):
        pltpu.async_copy(
            bufos[p],
            out_ref.at[pl.ds(base_b + c * _BPC, _BPC)],
            osems[p],
        )

    def wait_out(p):
        pltpu.make_async_copy(out_ref.at[pl.ds(0, _BPC)],
                              bufos[p], osems[p]).wait()

    for g in range(_NG):
        gather(g, g)

    def body(k, _):
        c0 = _NG * k
        for g in range(_NG):
            c = c0 + g
            p = g & 1
            wait_gather(g)
            pl.when(c >= _NO)(lambda: wait_out(p))
            _repack_chunk(bufgs[g], bufos[p])
            pl.when(c + _NG < _NCHUNK)(lambda: gather(c + _NG, g))
            out_copy(c, p)
        return ()

    lax.fori_loop(0, _NCHUNK // _NG, body, ())
    wait_out(0)
    wait_out(1)


def _sc_gather(table8, jidx):
    mesh = plsc.VectorSubcoreMesh(core_axis_name="c", subcore_axis_name="s")
    fn = functools.partial(
        pl.kernel,
        out_type=jax.ShapeDtypeStruct((_B, _T, _VOCAB), jnp.float32),
        mesh=mesh,
        scratch_types=[
            pltpu.VMEM((_NCHUNK, _CHUNK), jnp.int32),
            pltpu.VMEM((_CHUNK, _VPAD), jnp.float32),
            pltpu.VMEM((_CHUNK, _VPAD), jnp.float32),
            pltpu.VMEM((_CHUNK, _VPAD), jnp.float32),
            pltpu.VMEM((_CHUNK, _VPAD), jnp.float32),
            pltpu.VMEM((_BPC, _T, _VOCAB), jnp.float32),
            pltpu.VMEM((_BPC, _T, _VOCAB), jnp.float32),
            pltpu.SemaphoreType.DMA,
            pltpu.SemaphoreType.DMA,
            pltpu.SemaphoreType.DMA,
            pltpu.SemaphoreType.DMA,
            pltpu.SemaphoreType.DMA,
            pltpu.SemaphoreType.DMA,
        ],
        compiler_params=pltpu.CompilerParams(needs_layout_passes=False),
    )(_sc_gather_body)
    return fn(table8, jidx)


@jax.jit
def kernel(idx, embedding, positional_embedding, lm_head_w, lm_head_b):
    table8 = _build_table8(embedding, positional_embedding, lm_head_w,
                           lm_head_b)
    flat = idx.reshape(_NROWS).astype(jnp.int32)
    j = flat + _VOCAB * (jax.lax.iota(jnp.int32, _NROWS) % _T)
    jidx = j.reshape(_NW, _NCHUNK, _CHUNK)
    return _sc_gather(table8, jidx)
